# Initial kernel scaffold; baseline (speedup 1.0000x reference)
import jax
import jax.numpy as jnp
from jax.experimental import pallas as pl


def kernel(input_ids):
    raise NotImplementedError("write your pallas kernel here")


# speedup vs baseline: 1.2416x; 1.2416x over previous
import functools

import numpy as np
import jax
import jax.numpy as jnp
from jax import lax
from jax.experimental import pallas as pl
from jax.experimental.pallas import tpu as pltpu
from jax.experimental.pallas import tpu_sc as plsc


_U32 = np.uint32


def _np_rotl(x, r):
    x = x.astype(np.uint32)
    return ((x << _U32(r)) | (x >> _U32(32 - r))).astype(np.uint32)


def _np_threefry2x32(k0, k1, x0, x1):
    rot_a = (13, 15, 26, 6)
    rot_b = (17, 29, 16, 24)
    ks = (_U32(k0), _U32(k1), _U32(k0) ^ _U32(k1) ^ _U32(0x1BD11BDA))
    v0 = (np.asarray(x0, np.uint32) + ks[0]).astype(np.uint32)
    v1 = (np.asarray(x1, np.uint32) + ks[1]).astype(np.uint32)

    def rounds(v0, v1, rs):
        for r in rs:
            v0 = (v0 + v1).astype(np.uint32)
            v1 = _np_rotl(v1, r) ^ v0
        return v0, v1

    inj = ((ks[1], ks[2], 1), (ks[2], ks[0], 2), (ks[0], ks[1], 3),
           (ks[1], ks[2], 4), (ks[2], ks[0], 5))
    for i, (a, b, c) in enumerate(inj):
        v0, v1 = rounds(v0, v1, rot_a if i % 2 == 0 else rot_b)
        v0 = (v0 + a).astype(np.uint32)
        v1 = (v1 + b + _U32(c)).astype(np.uint32)
    return v0, v1


def _np_split(key, num):
    c1 = np.zeros(num, np.uint32)
    c2 = np.arange(num, dtype=np.uint32)
    b1, b2 = _np_threefry2x32(key[0], key[1], c1, c2)
    return np.stack([b1, b2], axis=1)


def _row_subkeys(batch, seed=42):
    base = np.array([0, seed], np.uint32)
    rowkeys = _np_split(base, batch)
    sub1 = np.empty((batch, 2), np.uint32)
    sub2 = np.empty((batch, 2), np.uint32)
    for r in range(batch):
        ks = _np_split(rowkeys[r], 2)
        key1, sub1[r] = ks[0], ks[1]
        ks2 = _np_split(key1, 2)
        sub2[r] = ks2[1]
    return sub1, sub2


_ROT_A = (13, 15, 26, 6)
_ROT_B = (17, 29, 16, 24)


def _tc_bits(k0, k1, n):
    ks2 = k0 ^ k1 ^ jnp.uint32(0x1BD11BDA)
    x1 = lax.broadcasted_iota(jnp.uint32, (k0.shape[0], n), 1)
    v0 = k0 + jnp.zeros_like(x1)
    v1 = x1 + k1

    def rounds(v0, v1, rs):
        for r in rs:
            v0 = v0 + v1
            v1 = ((v1 << r) | (v1 >> (32 - r))) ^ v0
        return v0, v1

    inj = ((k1, ks2, 1), (ks2, k0, 2), (k0, k1, 3), (k1, ks2, 4), (ks2, k0, 5))
    for i, (a, b, c) in enumerate(inj):
        v0, v1 = rounds(v0, v1, _ROT_A if i % 2 == 0 else _ROT_B)
        v0 = v0 + a
        v1 = v1 + b + jnp.uint32(c)
    return v0 ^ v1


def _rng_kernel(s1a_ref, s1b_ref, s2a_ref, s2b_ref, bits1_ref, bits2_ref):
    n = bits1_ref.shape[1]
    b1 = _tc_bits(s1a_ref[...], s1b_ref[...], n)
    b2 = _tc_bits(s2a_ref[...], s2b_ref[...], n)
    bits1_ref[...] = lax.bitcast_convert_type(b1, jnp.int32)
    bits2_ref[...] = lax.bitcast_convert_type(b2, jnp.int32)


def _make_bits(batch, n, interpret=False):
    sub1, sub2 = _row_subkeys(batch)
    s1a = jnp.asarray(sub1[:, 0:1])
    s1b = jnp.asarray(sub1[:, 1:2])
    s2a = jnp.asarray(sub2[:, 0:1])
    s2b = jnp.asarray(sub2[:, 1:2])
    rows_blk = 16
    grid = (batch // rows_blk,)
    kspec = pl.BlockSpec((rows_blk, 1), lambda i: (i, 0))
    bspec = pl.BlockSpec((rows_blk, n), lambda i: (i, 0))
    return pl.pallas_call(
        _rng_kernel,
        grid=grid,
        in_specs=[kspec] * 4,
        out_specs=[bspec, bspec],
        out_shape=[
            jax.ShapeDtypeStruct((batch, n), jnp.int32),
            jax.ShapeDtypeStruct((batch, n), jnp.int32),
        ],
        interpret=interpret,
    )(s1a, s1b, s2a, s2b)


_L = 16
_RAD = 256
_NH = _RAD * _L


def _sc_mask_kernel(batch, n, nf):
    nv = n // _L
    stride = nv
    mesh = plsc.VectorSubcoreMesh(core_axis_name="c", subcore_axis_name="s")
    nw = mesh.num_cores * mesh.num_subcores
    rows_per = batch // nw

    def body(bits1_hbm, bits2_hbm, out_hbm, kA, kB, b2, pA, pB, hist, sums,
             sflag, mrow):
        iota = lax.iota(jnp.int32, _L)
        ones = jnp.ones((_L,), jnp.int32)
        zeros = jnp.zeros((_L,), jnp.int32)
        wid = lax.axis_index("s") * mesh.num_cores + lax.axis_index("c")

        def digit_of(k, shift):
            return lax.shift_right_logical(k, jnp.full((_L,), shift, jnp.int32)) & 255

        def histogram(kin, shift):
            def zero(v, _):
                hist[pl.ds(v * _L, _L)] = zeros
                return 0
            lax.fori_loop(0, _NH // _L, zero, 0)

            def acc(t, _):
                src = iota * stride + t
                k = plsc.load_gather(kin, [src])
                hidx = digit_of(k, shift) * _L + iota
                plsc.addupdate_scatter(hist, [hidx], ones)
                return 0
            lax.fori_loop(0, nv, acc, 0)

        def scan_hist():
            def s1(g, _):
                base = (g * _L + iota) * _L
                a = zeros
                for j in range(_L):
                    a = a + plsc.load_gather(hist, [base + j])
                sums[pl.ds(g * _L, _L)] = a
                return 0
            lax.fori_loop(0, _RAD // _L, s1, 0)

            def s2(g, carry):
                v = sums[pl.ds(g * _L, _L)]
                incl = plsc.cumsum(v)
                sums[pl.ds(g * _L, _L)] = incl - v + carry
                return carry + jnp.sum(v)
            lax.fori_loop(0, _RAD // _L, s2, jnp.int32(0))

            def fixup(d, _):
                h = hist[pl.ds(d * _L, _L)]
                incl = plsc.cumsum(h)
                base = plsc.load_gather(sums, [iota * 0 + d])
                hist[pl.ds(d * _L, _L)] = incl - h + base
                return 0
            lax.fori_loop(0, _RAD, fixup, 0)

        def permute(kin, shift, kout=None, pin=None, pout=None, fused=False):
            def step(t, _):
                src = iota * stride + t
                k = plsc.load_gather(kin, [src])
                hidx = digit_of(k, shift) * _L + iota
                dest = plsc.load_gather(hist, [hidx])
                plsc.addupdate_scatter(hist, [hidx], ones)
                p = src if pin is None else plsc.load_gather(pin, [src])
                if fused:
                    s = plsc.load_gather(sflag, [dest])
                    plsc.store_scatter(mrow, [p], 1 - s)
                else:
                    if kout is not None:
                        plsc.store_scatter(kout, [dest], k)
                    plsc.store_scatter(pout, [dest], p)
                return 0
            lax.fori_loop(0, nv, step, 0)

        def radix_pass(kin, shift, **kw):
            histogram(kin, shift)
            scan_hist()
            permute(kin, shift, **kw)

        def row_body(j, _):
            row = wid * rows_per + j
            pltpu.sync_copy(bits1_hbm.at[row], kA)
            pltpu.sync_copy(bits2_hbm.at[row], b2)

            radix_pass(b2, 0, kout=kB, pin=None, pout=pB)
            radix_pass(kB, 8, kout=b2, pin=pB, pout=pA)
            radix_pass(b2, 16, kout=kB, pin=pA, pout=pB)
            radix_pass(kB, 24, kout=None, pin=pB, pout=pA)

            def zflag(t, _):
                sflag[pl.ds(t * _L, _L)] = zeros
                return 0
            lax.fori_loop(0, nv, zflag, 0)

            def setflag(t, _):
                q = t * _L + iota
                pos = plsc.load_gather(pA, [q])
                plsc.store_scatter(sflag, [pos], ones, mask=q < nf)
                return 0
            lax.fori_loop(0, (nf + _L - 1) // _L, setflag, 0)

            radix_pass(kA, 0, kout=kB, pin=None, pout=pB)
            radix_pass(kB, 8, kout=kA, pin=pB, pout=pA)
            radix_pass(kA, 16, kout=kB, pin=pA, pout=pB)
            radix_pass(kB, 24, pin=pB, fused=True)

            pltpu.sync_copy(mrow, out_hbm.at[row])
            return 0

        lax.fori_loop(0, rows_per, row_body, 0)

    vm = lambda shape: pltpu.VMEM(shape, jnp.int32)
    return pl.kernel(
        body,
        out_type=jax.ShapeDtypeStruct((batch, n), jnp.int32),
        mesh=mesh,
        compiler_params=pltpu.CompilerParams(needs_layout_passes=False),
        scratch_types=[
            vm((n,)),
            vm((n,)),
            vm((n,)),
            vm((n,)),
            vm((n,)),
            vm((_NH,)),
            vm((_RAD,)),
            vm((n,)),
            vm((n,)),
        ],
    )


def kernel(input_ids):
    batch, n = input_ids.shape
    nf = int(n * 0.1)
    bits1, bits2 = _make_bits(batch, n)
    mask_i32 = _sc_mask_kernel(batch, n, nf)(bits1, bits2)
    return mask_i32.astype(jnp.bool_)


# speedup vs baseline: 1.7766x; 1.4309x over previous
import functools

import numpy as np
import jax
import jax.numpy as jnp
from jax import lax
from jax.experimental import pallas as pl
from jax.experimental.pallas import tpu as pltpu
from jax.experimental.pallas import tpu_sc as plsc


_U32 = np.uint32


def _np_rotl(x, r):
    x = x.astype(np.uint32)
    return ((x << _U32(r)) | (x >> _U32(32 - r))).astype(np.uint32)


def _np_threefry2x32(k0, k1, x0, x1):
    rot_a = (13, 15, 26, 6)
    rot_b = (17, 29, 16, 24)
    ks = (_U32(k0), _U32(k1), _U32(k0) ^ _U32(k1) ^ _U32(0x1BD11BDA))
    v0 = (np.asarray(x0, np.uint32) + ks[0]).astype(np.uint32)
    v1 = (np.asarray(x1, np.uint32) + ks[1]).astype(np.uint32)

    def rounds(v0, v1, rs):
        for r in rs:
            v0 = (v0 + v1).astype(np.uint32)
            v1 = _np_rotl(v1, r) ^ v0
        return v0, v1

    inj = ((ks[1], ks[2], 1), (ks[2], ks[0], 2), (ks[0], ks[1], 3),
           (ks[1], ks[2], 4), (ks[2], ks[0], 5))
    for i, (a, b, c) in enumerate(inj):
        v0, v1 = rounds(v0, v1, rot_a if i % 2 == 0 else rot_b)
        v0 = (v0 + a).astype(np.uint32)
        v1 = (v1 + b + _U32(c)).astype(np.uint32)
    return v0, v1


def _np_split(key, num):
    c1 = np.zeros(num, np.uint32)
    c2 = np.arange(num, dtype=np.uint32)
    b1, b2 = _np_threefry2x32(key[0], key[1], c1, c2)
    return np.stack([b1, b2], axis=1)


def _row_subkeys(batch, seed=42):
    base = np.array([0, seed], np.uint32)
    rowkeys = _np_split(base, batch)
    sub1 = np.empty((batch, 2), np.uint32)
    sub2 = np.empty((batch, 2), np.uint32)
    for r in range(batch):
        ks = _np_split(rowkeys[r], 2)
        key1, sub1[r] = ks[0], ks[1]
        ks2 = _np_split(key1, 2)
        sub2[r] = ks2[1]
    return sub1, sub2


_ROT_A = (13, 15, 26, 6)
_ROT_B = (17, 29, 16, 24)


def _tc_bits(k0, k1, n):
    ks2 = k0 ^ k1 ^ jnp.uint32(0x1BD11BDA)
    x1 = lax.broadcasted_iota(jnp.uint32, (k0.shape[0], n), 1)
    v0 = k0 + jnp.zeros_like(x1)
    v1 = x1 + k1

    def rounds(v0, v1, rs):
        for r in rs:
            v0 = v0 + v1
            v1 = ((v1 << r) | (v1 >> (32 - r))) ^ v0
        return v0, v1

    inj = ((k1, ks2, 1), (ks2, k0, 2), (k0, k1, 3), (k1, ks2, 4), (ks2, k0, 5))
    for i, (a, b, c) in enumerate(inj):
        v0, v1 = rounds(v0, v1, _ROT_A if i % 2 == 0 else _ROT_B)
        v0 = v0 + a
        v1 = v1 + b + jnp.uint32(c)
    return v0 ^ v1


def _rng_kernel(s1a_ref, s1b_ref, s2a_ref, s2b_ref, bits1_ref, bits2_ref):
    n = bits1_ref.shape[1]
    b1 = _tc_bits(s1a_ref[...], s1b_ref[...], n)
    b2 = _tc_bits(s2a_ref[...], s2b_ref[...], n)
    bits1_ref[...] = lax.bitcast_convert_type(b1, jnp.int32)
    bits2_ref[...] = lax.bitcast_convert_type(b2, jnp.int32)


def _make_bits(batch, n, interpret=False):
    sub1, sub2 = _row_subkeys(batch)
    s1a = jnp.asarray(sub1[:, 0:1])
    s1b = jnp.asarray(sub1[:, 1:2])
    s2a = jnp.asarray(sub2[:, 0:1])
    s2b = jnp.asarray(sub2[:, 1:2])
    rows_blk = 16
    grid = (batch // rows_blk,)
    kspec = pl.BlockSpec((rows_blk, 1), lambda i: (i, 0))
    bspec = pl.BlockSpec((rows_blk, n), lambda i: (i, 0))
    return pl.pallas_call(
        _rng_kernel,
        grid=grid,
        in_specs=[kspec] * 4,
        out_specs=[bspec, bspec],
        out_shape=[
            jax.ShapeDtypeStruct((batch, n), jnp.int32),
            jax.ShapeDtypeStruct((batch, n), jnp.int32),
        ],
        interpret=interpret,
    )(s1a, s1b, s2a, s2b)


_L = 16
_RAD = 256
_NH = _RAD * _L


def _sc_mask_kernel(batch, n, nf):
    nv = n // _L
    stride = nv
    mesh = plsc.VectorSubcoreMesh(core_axis_name="c", subcore_axis_name="s")
    nw = mesh.num_cores * mesh.num_subcores
    rows_per = batch // nw

    def body(bits1_hbm, bits2_hbm, out_hbm, kA, kB, b2, pA, pB, hist, sums,
             sflag, mrow):
        iota = lax.iota(jnp.int32, _L)
        ones = jnp.ones((_L,), jnp.int32)
        zeros = jnp.zeros((_L,), jnp.int32)
        wid = lax.axis_index("s") * mesh.num_cores + lax.axis_index("c")

        def digit_of(k, shift):
            return lax.shift_right_logical(k, jnp.full((_L,), shift, jnp.int32)) & 255

        def histogram(kin, shift):
            @plsc.parallel_loop(0, _NH // _L, unroll=8)
            def _zero(v):
                hist[pl.ds(v * _L, _L)] = zeros

            @plsc.parallel_loop(0, nv, unroll=8)
            def _acc(t):
                src = iota * stride + t
                k = plsc.load_gather(kin, [src])
                hidx = digit_of(k, shift) * _L + iota
                plsc.addupdate_scatter(hist, [hidx], ones)

        def scan_hist():
            @plsc.parallel_loop(0, _RAD // _L, unroll=4)
            def _s1(g):
                base = (g * _L + iota) * _L
                a = zeros
                for j in range(_L):
                    a = a + plsc.load_gather(hist, [base + j])
                sums[pl.ds(g * _L, _L)] = a

            @plsc.parallel_loop(0, _RAD // _L, unroll=4, carry=jnp.int32(0))
            def _s2(g, carry):
                v = sums[pl.ds(g * _L, _L)]
                incl = plsc.cumsum(v)
                sums[pl.ds(g * _L, _L)] = incl - v + carry
                return carry + jnp.sum(v)

            @plsc.parallel_loop(0, _RAD, unroll=8)
            def _fixup(d):
                h = hist[pl.ds(d * _L, _L)]
                incl = plsc.cumsum(h)
                base = plsc.load_gather(sums, [iota * 0 + d])
                hist[pl.ds(d * _L, _L)] = incl - h + base

        def permute(kin, shift, kout=None, pin=None, pout=None, fused=False):
            def step(t, _):
                src = iota * stride + t
                k = plsc.load_gather(kin, [src])
                hidx = digit_of(k, shift) * _L + iota
                dest = plsc.load_gather(hist, [hidx])
                plsc.addupdate_scatter(hist, [hidx], ones)
                p = src if pin is None else plsc.load_gather(pin, [src])
                if fused:
                    s = plsc.load_gather(sflag, [dest])
                    plsc.store_scatter(mrow, [p], 1 - s)
                else:
                    if kout is not None:
                        plsc.store_scatter(kout, [dest], k)
                    plsc.store_scatter(pout, [dest], p)
                return 0
            lax.fori_loop(0, nv, step, 0, unroll=8)

        def radix_pass(kin, shift, **kw):
            histogram(kin, shift)
            scan_hist()
            permute(kin, shift, **kw)

        def row_body(j, _):
            row = wid * rows_per + j
            pltpu.sync_copy(bits1_hbm.at[row], kA)
            pltpu.sync_copy(bits2_hbm.at[row], b2)

            radix_pass(b2, 0, kout=kB, pin=None, pout=pB)
            radix_pass(kB, 8, kout=b2, pin=pB, pout=pA)
            radix_pass(b2, 16, kout=kB, pin=pA, pout=pB)
            radix_pass(kB, 24, kout=None, pin=pB, pout=pA)

            @plsc.parallel_loop(0, nv, unroll=8)
            def _zflag(t):
                sflag[pl.ds(t * _L, _L)] = zeros

            @plsc.parallel_loop(0, (nf + _L - 1) // _L, unroll=4)
            def _setflag(t):
                q = t * _L + iota
                pos = plsc.load_gather(pA, [q])
                plsc.store_scatter(sflag, [pos], ones, mask=q < nf)

            radix_pass(kA, 0, kout=kB, pin=None, pout=pB)
            radix_pass(kB, 8, kout=kA, pin=pB, pout=pA)
            radix_pass(kA, 16, kout=kB, pin=pA, pout=pB)
            radix_pass(kB, 24, pin=pB, fused=True)

            pltpu.sync_copy(mrow, out_hbm.at[row])
            return 0

        lax.fori_loop(0, rows_per, row_body, 0)

    vm = lambda shape: pltpu.VMEM(shape, jnp.int32)
    return pl.kernel(
        body,
        out_type=jax.ShapeDtypeStruct((batch, n), jnp.int32),
        mesh=mesh,
        compiler_params=pltpu.CompilerParams(needs_layout_passes=False),
        scratch_types=[
            vm((n,)),
            vm((n,)),
            vm((n,)),
            vm((n,)),
            vm((n,)),
            vm((_NH,)),
            vm((_RAD,)),
            vm((n,)),
            vm((n,)),
        ],
    )


def kernel(input_ids):
    batch, n = input_ids.shape
    nf = int(n * 0.1)
    bits1, bits2 = _make_bits(batch, n)
    mask_i32 = _sc_mask_kernel(batch, n, nf)(bits1, bits2)
    return mask_i32.astype(jnp.bool_)


# speedup vs baseline: 2.9216x; 1.6445x over previous
import functools

import numpy as np
import jax
import jax.numpy as jnp
from jax import lax
from jax.experimental import pallas as pl
from jax.experimental.pallas import tpu as pltpu
from jax.experimental.pallas import tpu_sc as plsc


_U32 = np.uint32


def _np_rotl(x, r):
    x = x.astype(np.uint32)
    return ((x << _U32(r)) | (x >> _U32(32 - r))).astype(np.uint32)


def _np_threefry2x32(k0, k1, x0, x1):
    rot_a = (13, 15, 26, 6)
    rot_b = (17, 29, 16, 24)
    ks = (_U32(k0), _U32(k1), _U32(k0) ^ _U32(k1) ^ _U32(0x1BD11BDA))
    v0 = (np.asarray(x0, np.uint32) + ks[0]).astype(np.uint32)
    v1 = (np.asarray(x1, np.uint32) + ks[1]).astype(np.uint32)

    def rounds(v0, v1, rs):
        for r in rs:
            v0 = (v0 + v1).astype(np.uint32)
            v1 = _np_rotl(v1, r) ^ v0
        return v0, v1

    inj = ((ks[1], ks[2], 1), (ks[2], ks[0], 2), (ks[0], ks[1], 3),
           (ks[1], ks[2], 4), (ks[2], ks[0], 5))
    for i, (a, b, c) in enumerate(inj):
        v0, v1 = rounds(v0, v1, rot_a if i % 2 == 0 else rot_b)
        v0 = (v0 + a).astype(np.uint32)
        v1 = (v1 + b + _U32(c)).astype(np.uint32)
    return v0, v1


def _np_split(key, num):
    c1 = np.zeros(num, np.uint32)
    c2 = np.arange(num, dtype=np.uint32)
    b1, b2 = _np_threefry2x32(key[0], key[1], c1, c2)
    return np.stack([b1, b2], axis=1)


def _row_subkeys(batch, seed=42):
    base = np.array([0, seed], np.uint32)
    rowkeys = _np_split(base, batch)
    sub1 = np.empty((batch, 2), np.uint32)
    sub2 = np.empty((batch, 2), np.uint32)
    for r in range(batch):
        ks = _np_split(rowkeys[r], 2)
        key1, sub1[r] = ks[0], ks[1]
        ks2 = _np_split(key1, 2)
        sub2[r] = ks2[1]
    return sub1, sub2


_ROT_A = (13, 15, 26, 6)
_ROT_B = (17, 29, 16, 24)


def _tc_bits(k0, k1, n):
    ks2 = k0 ^ k1 ^ jnp.uint32(0x1BD11BDA)
    x1 = lax.broadcasted_iota(jnp.uint32, (k0.shape[0], n), 1)
    v0 = k0 + jnp.zeros_like(x1)
    v1 = x1 + k1

    def rounds(v0, v1, rs):
        for r in rs:
            v0 = v0 + v1
            v1 = ((v1 << r) | (v1 >> (32 - r))) ^ v0
        return v0, v1

    inj = ((k1, ks2, 1), (ks2, k0, 2), (k0, k1, 3), (k1, ks2, 4), (ks2, k0, 5))
    for i, (a, b, c) in enumerate(inj):
        v0, v1 = rounds(v0, v1, _ROT_A if i % 2 == 0 else _ROT_B)
        v0 = v0 + a
        v1 = v1 + b + jnp.uint32(c)
    return v0 ^ v1


def _rng_kernel(s1a_ref, s1b_ref, s2a_ref, s2b_ref, bits1_ref, bits2_ref):
    n = bits1_ref.shape[1]
    b1 = _tc_bits(s1a_ref[...], s1b_ref[...], n)
    b2 = _tc_bits(s2a_ref[...], s2b_ref[...], n)
    bits1_ref[...] = lax.bitcast_convert_type(b1, jnp.int32)
    bits2_ref[...] = lax.bitcast_convert_type(b2, jnp.int32)


def _make_bits(batch, n, interpret=False):
    sub1, sub2 = _row_subkeys(batch)
    s1a = jnp.asarray(sub1[:, 0:1])
    s1b = jnp.asarray(sub1[:, 1:2])
    s2a = jnp.asarray(sub2[:, 0:1])
    s2b = jnp.asarray(sub2[:, 1:2])
    rows_blk = 16
    grid = (batch // rows_blk,)
    kspec = pl.BlockSpec((rows_blk, 1), lambda i: (i, 0))
    bspec = pl.BlockSpec((rows_blk, n), lambda i: (i, 0))
    return pl.pallas_call(
        _rng_kernel,
        grid=grid,
        in_specs=[kspec] * 4,
        out_specs=[bspec, bspec],
        out_shape=[
            jax.ShapeDtypeStruct((batch, n), jnp.int32),
            jax.ShapeDtypeStruct((batch, n), jnp.int32),
        ],
        interpret=interpret,
    )(s1a, s1b, s2a, s2b)


_L = 16
_RAD = 256
_NH = _RAD * _L


def _sc_mask_kernel(batch, n, nf):
    nv = n // _L
    stride = nv
    mesh = plsc.VectorSubcoreMesh(core_axis_name="c", subcore_axis_name="s")
    nw = mesh.num_cores * mesh.num_subcores
    rows_per = batch // nw

    def body(bits1_hbm, bits2_hbm, out_hbm, kA, kB, b2, pA, pB, hist, sums,
             sflag, mrow):
        iota = lax.iota(jnp.int32, _L)
        ones = jnp.ones((_L,), jnp.int32)
        zeros = jnp.zeros((_L,), jnp.int32)
        wid = lax.axis_index("s") * mesh.num_cores + lax.axis_index("c")

        def digit_of(k, shift):
            return lax.shift_right_logical(k, jnp.full((_L,), shift, jnp.int32)) & 255

        def histogram(kin, shift):
            @plsc.parallel_loop(0, _NH // _L, unroll=8)
            def _zero(v):
                hist[pl.ds(v * _L, _L)] = zeros

            @plsc.parallel_loop(0, nv, unroll=8)
            def _acc(t):
                src = iota * stride + t
                k = plsc.load_gather(kin, [src])
                hidx = digit_of(k, shift) * _L + iota
                plsc.addupdate_scatter(hist, [hidx], ones)

        def scan_hist():
            @plsc.parallel_loop(0, _RAD // _L, unroll=4)
            def _s1(g):
                base = (g * _L + iota) * _L
                a = zeros
                for j in range(_L):
                    a = a + plsc.load_gather(hist, [base + j])
                sums[pl.ds(g * _L, _L)] = a

            @plsc.parallel_loop(0, _RAD // _L, unroll=4, carry=jnp.int32(0))
            def _s2(g, carry):
                v = sums[pl.ds(g * _L, _L)]
                incl = plsc.cumsum(v)
                sums[pl.ds(g * _L, _L)] = incl - v + carry
                return carry + jnp.sum(v)

            @plsc.parallel_loop(0, _RAD, unroll=8)
            def _fixup(d):
                h = hist[pl.ds(d * _L, _L)]
                incl = plsc.cumsum(h)
                base = plsc.load_gather(sums, [iota * 0 + d])
                hist[pl.ds(d * _L, _L)] = incl - h + base

        def permute(kin, shift, kout=None, pin=None, pout=None, fused=False):
            def step(t, _):
                src = iota * stride + t
                k = plsc.load_gather(kin, [src])
                hidx = digit_of(k, shift) * _L + iota
                dest = plsc.load_gather(hist, [hidx])
                plsc.addupdate_scatter(hist, [hidx], ones)
                p = src if pin is None else plsc.load_gather(pin, [src])
                if fused:
                    s = plsc.load_gather(sflag, [dest])
                    plsc.store_scatter(mrow, [p], 1 - s)
                else:
                    if kout is not None:
                        plsc.store_scatter(kout, [dest], k)
                    plsc.store_scatter(pout, [dest], p)
                return 0
            lax.fori_loop(0, nv, step, 0, unroll=8)

        def radix_pass(kin, shift, **kw):
            histogram(kin, shift)
            scan_hist()
            permute(kin, shift, **kw)

        minint = jnp.int32(-2147483648)

        def row_body(j, _):
            row = wid * rows_per + j
            pltpu.sync_copy(bits1_hbm.at[row], kA)
            pltpu.sync_copy(bits2_hbm.at[row], b2)

            pref = jnp.int32(0)
            r_rem = jnp.int32(nf - 1)
            for level in range(4):
                shift = 24 - 8 * level

                @plsc.parallel_loop(0, _NH // _L, unroll=8)
                def _zero(v):
                    hist[pl.ds(v * _L, _L)] = zeros

                @plsc.parallel_loop(0, nv, unroll=8)
                def _h(t, shift=shift, level=level, pref=pref):
                    k = b2[pl.ds(t * _L, _L)]
                    d = digit_of(k, shift)
                    if level == 0:
                        plsc.addupdate_scatter(hist, [d * _L + iota], ones)
                    else:
                        act = lax.shift_right_logical(
                            k, jnp.full((_L,), shift + 8, jnp.int32)) == pref
                        plsc.addupdate_scatter(hist, [d * _L + iota], ones,
                                               mask=act)

                @plsc.parallel_loop(0, _RAD // _L, unroll=4)
                def _s1(g):
                    base = (g * _L + iota) * _L
                    a = zeros
                    for jj in range(_L):
                        a = a + plsc.load_gather(hist, [base + jj])
                    sums[pl.ds(g * _L, _L)] = a

                @plsc.parallel_loop(0, _RAD // _L, unroll=4,
                                    carry=jnp.int32(0))
                def _s2(g, carry):
                    v = sums[pl.ds(g * _L, _L)]
                    incl = plsc.cumsum(v)
                    sums[pl.ds(g * _L, _L)] = incl - v + carry
                    return carry + jnp.sum(v)

                @plsc.parallel_loop(0, _RAD // _L, unroll=4, carry=zeros)
                def _loc(g, acc):
                    e = sums[pl.ds(g * _L, _L)]
                    return acc + (e <= r_rem).astype(jnp.int32)

                bdig = jnp.sum(_loc) - 1
                cbv = plsc.load_gather(sums, [iota * 0 + bdig])
                r_rem = r_rem - jnp.max(cbv)
                pref = (pref << 8) | bdig

            t2 = pref
            mth = r_rem + 1

            @plsc.parallel_loop(0, nv, unroll=4, carry=jnp.int32(0))
            def _sf(t, eqc):
                k = b2[pl.ds(t * _L, _L)]
                lt = (k ^ minint) < (t2 ^ minint)
                eq = k == t2
                eqi = eq.astype(jnp.int32)
                eqpos = plsc.cumsum(eqi) - eqi + eqc
                fl = lt | (eq & (eqpos < mth))
                sflag[pl.ds(t * _L, _L)] = fl.astype(jnp.int32)
                return eqc + jnp.sum(eqi)

            radix_pass(kA, 0, kout=kB, pin=None, pout=pB)
            radix_pass(kB, 8, kout=kA, pin=pB, pout=pA)
            radix_pass(kA, 16, kout=kB, pin=pA, pout=pB)
            radix_pass(kB, 24, pin=pB, fused=True)

            pltpu.sync_copy(mrow, out_hbm.at[row])
            return 0

        lax.fori_loop(0, rows_per, row_body, 0)

    vm = lambda shape: pltpu.VMEM(shape, jnp.int32)
    return pl.kernel(
        body,
        out_type=jax.ShapeDtypeStruct((batch, n), jnp.int32),
        mesh=mesh,
        compiler_params=pltpu.CompilerParams(needs_layout_passes=False),
        scratch_types=[
            vm((n,)),
            vm((n,)),
            vm((n,)),
            vm((n,)),
            vm((n,)),
            vm((_NH,)),
            vm((_RAD,)),
            vm((n,)),
            vm((n,)),
        ],
    )


def kernel(input_ids):
    batch, n = input_ids.shape
    nf = int(n * 0.1)
    bits1, bits2 = _make_bits(batch, n)
    mask_i32 = _sc_mask_kernel(batch, n, nf)(bits1, bits2)
    return mask_i32.astype(jnp.bool_)
